# baseline (speedup 1.0000x reference)
import functools

import jax
import jax.numpy as jnp
from jax import lax
from jax.experimental import pallas as pl
from jax.experimental.pallas import tpu as pltpu
from jax.experimental.pallas import tpu_sc as plsc

B = 16384
F = 26
V = 100000
D = 64
H1 = 256
H2 = 128

NC = 2
NS = 16
NW = NC * NS
SPW = B // NW
SPG = 4
RPG = SPG * F
NG = SPW // SPG
NBUF = 2


def _sc_bi_body(flatx_hbm, table_hbm, bi_hbm, idx_v, rows_v, out_v, sem0, sem1):
    wid = lax.axis_index("s") * NC + lax.axis_index("c")
    pltpu.sync_copy(flatx_hbm.at[wid], idx_v)

    sems = (sem0, sem1)

    for b in range(NBUF):
        pltpu.async_copy(table_hbm.at[idx_v.at[b]], rows_v.at[b], sems[b])

    def outer(gg, carry):
        for b in range(NBUF):
            g = gg * NBUF + b
            pltpu.make_async_copy(
                table_hbm.at[idx_v.at[g]], rows_v.at[b], sems[b]
            ).wait()
            for si in range(SPG):
                row0 = si * F
                for c in range(D // 16):
                    sl = pl.ds(c * 16, 16)
                    v = rows_v[b, row0, sl]
                    acc = v
                    accsq = v * v
                    for f in range(1, F):
                        v = rows_v[b, row0 + f, sl]
                        acc = acc + v
                        accsq = accsq + v * v
                    out_v[g * SPG + si, sl] = 0.5 * (acc * acc - accsq)
            nxt = g + NBUF

            @pl.when(nxt < NG)
            def _():
                pltpu.async_copy(table_hbm.at[idx_v.at[nxt]], rows_v.at[b], sems[b])

        return carry

    lax.fori_loop(0, NG // NBUF, outer, 0)

    pltpu.sync_copy(out_v, bi_hbm.at[pl.ds(wid * SPW, SPW)])


_sc_bi = functools.partial(
    pl.kernel,
    mesh=plsc.VectorSubcoreMesh(core_axis_name="c", subcore_axis_name="s"),
    compiler_params=pltpu.CompilerParams(use_tc_tiling_on_sc=False),
    out_type=jax.ShapeDtypeStruct((B, D), jnp.float32),
    scratch_types=[
        pltpu.VMEM((NG, RPG), jnp.int32),
        pltpu.VMEM((NBUF, RPG, D), jnp.float32),
        pltpu.VMEM((SPW, D), jnp.float32),
        pltpu.SemaphoreType.DMA,
        pltpu.SemaphoreType.DMA,
    ],
)(_sc_bi_body)


BT = 1024


def _mlp_body(bi, xf, w1, b1, w2, b2, w3, wl, c0, out):
    h = jnp.maximum(
        jnp.dot(bi[...], w1[...], preferred_element_type=jnp.float32) + b1[...], 0.0
    )
    h = jnp.maximum(
        jnp.dot(h, w2[...], preferred_element_type=jnp.float32) + b2[...], 0.0
    )
    deep = jnp.dot(h, w3[...], preferred_element_type=jnp.float32)
    lin = jnp.dot(xf[...], wl[...], preferred_element_type=jnp.float32)
    out[...] = jax.nn.sigmoid(deep + lin + c0[...])


def _mlp_call(bi, xf, w1, b1, w2, b2, w3, wl, c0):
    grid = (B // BT,)
    return pl.pallas_call(
        _mlp_body,
        grid=grid,
        in_specs=[
            pl.BlockSpec((BT, D), lambda i: (i, 0)),
            pl.BlockSpec((BT, F), lambda i: (i, 0)),
            pl.BlockSpec((D, H1), lambda i: (0, 0)),
            pl.BlockSpec((1, H1), lambda i: (0, 0)),
            pl.BlockSpec((H1, H2), lambda i: (0, 0)),
            pl.BlockSpec((1, H2), lambda i: (0, 0)),
            pl.BlockSpec((H2, 1), lambda i: (0, 0)),
            pl.BlockSpec((F, 1), lambda i: (0, 0)),
            pl.BlockSpec((1, 1), lambda i: (0, 0)),
        ],
        out_specs=pl.BlockSpec((BT, 1), lambda i: (i, 0)),
        out_shape=jax.ShapeDtypeStruct((B, 1), jnp.float32),
    )(bi, xf, w1, b1, w2, b2, w3, wl, c0)


def kernel(x, tables, Wl, bl, W1, b1, W2, b2, W3, b3):
    x = x.astype(jnp.int32)
    offs = (jnp.arange(F, dtype=jnp.int32) * V)[None, :]
    flatx = (x + offs).reshape(NW, NG, RPG)
    table2d = tables.reshape(F * V, D)
    bi = _sc_bi(flatx, table2d)
    xf = x.astype(jnp.float32)
    c0 = (bl + b3).reshape(1, 1)
    return _mlp_call(
        bi,
        xf,
        W1,
        b1.reshape(1, H1),
        W2,
        b2.reshape(1, H2),
        W3,
        Wl,
        c0,
    )


# speedup vs baseline: 1.0815x; 1.0815x over previous
import functools

import jax
import jax.numpy as jnp
from jax import lax
from jax.experimental import pallas as pl
from jax.experimental.pallas import tpu as pltpu
from jax.experimental.pallas import tpu_sc as plsc

B = 16384
F = 26
V = 100000
D = 64
H1 = 256
H2 = 128

NC = 2
NS = 16
NW = NC * NS
SPW = B // NW
ST = 32
NT = SPW // ST
NBUF = 2


def _sc_bi_body(xt_hbm, tables_hbm, bi_hbm, idx_v, buf_v, out_v, sem0, sem1):
    wid = lax.axis_index("s") * NC + lax.axis_index("c")
    base = wid * SPW
    pltpu.sync_copy(xt_hbm.at[:, pl.ds(base, SPW)], idx_v)

    sems = (sem0, sem1)

    def fire(t, b):
        for f in range(F):
            pltpu.async_copy(
                tables_hbm.at[f].at[idx_v.at[f, pl.ds(t * ST, ST)]],
                buf_v.at[b, f],
                sems[b],
            )

    def drain(t, b):
        for f in range(F):
            pltpu.make_async_copy(
                tables_hbm.at[f].at[idx_v.at[f, pl.ds(t * ST, ST)]],
                buf_v.at[b, f],
                sems[b],
            ).wait()

    for b in range(NBUF):
        fire(b, b)

    def outer(tt, carry):
        for b in range(NBUF):
            t = tt * NBUF + b
            drain(t, b)

            def compute(si, c2):
                for c in range(D // 16):
                    sl = pl.ds(c * 16, 16)
                    v = buf_v[b, 0, si, sl]
                    acc = v
                    accsq = v * v
                    for f in range(1, F):
                        v = buf_v[b, f, si, sl]
                        acc = acc + v
                        accsq = accsq + v * v
                    out_v[si, sl] = 0.5 * (acc * acc - accsq)
                return c2

            lax.fori_loop(0, ST, compute, 0)

            pltpu.sync_copy(out_v, bi_hbm.at[pl.ds(base + t * ST, ST)])
            nxt = t + NBUF

            @pl.when(nxt < NT)
            def _():
                fire(nxt, b)

        return carry

    lax.fori_loop(0, NT // NBUF, outer, 0)


_sc_bi = functools.partial(
    pl.kernel,
    mesh=plsc.VectorSubcoreMesh(core_axis_name="c", subcore_axis_name="s"),
    compiler_params=pltpu.CompilerParams(use_tc_tiling_on_sc=False),
    out_type=jax.ShapeDtypeStruct((B, D), jnp.float32),
    scratch_types=[
        pltpu.VMEM((F, SPW), jnp.int32),
        pltpu.VMEM((NBUF, F, ST, D), jnp.float32),
        pltpu.VMEM((ST, D), jnp.float32),
        pltpu.SemaphoreType.DMA,
        pltpu.SemaphoreType.DMA,
    ],
)(_sc_bi_body)


BT = 1024


def _mlp_body(bi, xf, w1, b1, w2, b2, w3, wl, c0, out):
    h = jnp.maximum(
        jnp.dot(bi[...], w1[...], preferred_element_type=jnp.float32) + b1[...], 0.0
    )
    h = jnp.maximum(
        jnp.dot(h, w2[...], preferred_element_type=jnp.float32) + b2[...], 0.0
    )
    deep = jnp.dot(h, w3[...], preferred_element_type=jnp.float32)
    lin = jnp.dot(xf[...], wl[...], preferred_element_type=jnp.float32)
    out[...] = jax.nn.sigmoid(deep + lin + c0[...])


def _mlp_call(bi, xf, w1, b1, w2, b2, w3, wl, c0):
    grid = (B // BT,)
    return pl.pallas_call(
        _mlp_body,
        grid=grid,
        in_specs=[
            pl.BlockSpec((BT, D), lambda i: (i, 0)),
            pl.BlockSpec((BT, F), lambda i: (i, 0)),
            pl.BlockSpec((D, H1), lambda i: (0, 0)),
            pl.BlockSpec((1, H1), lambda i: (0, 0)),
            pl.BlockSpec((H1, H2), lambda i: (0, 0)),
            pl.BlockSpec((1, H2), lambda i: (0, 0)),
            pl.BlockSpec((H2, 1), lambda i: (0, 0)),
            pl.BlockSpec((F, 1), lambda i: (0, 0)),
            pl.BlockSpec((1, 1), lambda i: (0, 0)),
        ],
        out_specs=pl.BlockSpec((BT, 1), lambda i: (i, 0)),
        out_shape=jax.ShapeDtypeStruct((B, 1), jnp.float32),
    )(bi, xf, w1, b1, w2, b2, w3, wl, c0)


def kernel(x, tables, Wl, bl, W1, b1, W2, b2, W3, b3):
    x = x.astype(jnp.int32)
    xt = x.T
    bi = _sc_bi(xt, tables)
    xf = x.astype(jnp.float32)
    c0 = (bl + b3).reshape(1, 1)
    return _mlp_call(
        bi,
        xf,
        W1,
        b1.reshape(1, H1),
        W2,
        b2.reshape(1, H2),
        W3,
        Wl,
        c0,
    )


# speedup vs baseline: 1.4339x; 1.3259x over previous
import functools

import jax
import jax.numpy as jnp
from jax import lax
from jax.experimental import pallas as pl
from jax.experimental.pallas import tpu as pltpu
from jax.experimental.pallas import tpu_sc as plsc

B = 16384
F = 26
V = 100000
D = 64
H1 = 256
H2 = 128

NC = 2
NS = 16
NW = NC * NS
SPW = B // NW
CH = 64
NBUF = 2


def _sc_bi_body(x_hbm, tables_hbm, bi_hbm, idx_v, buf_v, out_v, sem0, sem1):
    wid = lax.axis_index("s") * NC + lax.axis_index("c")
    base = pl.multiple_of(wid * SPW, SPW)

    sems = (sem0, sem1)

    pltpu.sync_copy(
        x_hbm.at[pl.ds(pl.multiple_of(base * F, 8), SPW * F)],
        idx_v.at[pl.ds(0, SPW * F)],
    )

    def fire(s, b):
        va = idx_v[pl.ds(s * F, 16)]
        vb = idx_v[pl.ds(s * F + 16, 16)]
        for f in range(F):
            lane = va[f] if f < 16 else vb[f - 16]
            vi = pl.multiple_of(lane, 8)
            pltpu.async_copy(
                tables_hbm.at[f, pl.ds(vi, 1), :], buf_v.at[b, f], sems[b]
            )

    def drain(b):
        for f in range(F):
            pltpu.make_async_copy(
                tables_hbm.at[f, pl.ds(0, 1), :], buf_v.at[b, f], sems[b]
            ).wait()

    fire(0, 0)

    def outer(ss, carry):
        for b in range(NBUF):
            s = ss * NBUF + b
            nxt = s + 1

            @pl.when(nxt < SPW)
            def _():
                fire(nxt, b ^ 1)

            drain(b)
            for c in range(D // 16):
                sl = pl.ds(c * 16, 16)
                v = buf_v[b, 0, 0, sl]
                acc = v
                accsq = v * v
                for f in range(1, F):
                    v = buf_v[b, f, 0, sl]
                    acc = acc + v
                    accsq = accsq + v * v
                out_v[s, sl] = 0.5 * (acc * acc - accsq)

        return carry

    lax.fori_loop(0, SPW // NBUF, outer, 0)

    pltpu.sync_copy(out_v, bi_hbm.at[pl.ds(pl.multiple_of(base, SPW), SPW)])


_sc_bi = functools.partial(
    pl.kernel,
    mesh=plsc.VectorSubcoreMesh(core_axis_name="c", subcore_axis_name="s"),
    compiler_params=pltpu.CompilerParams(use_tc_tiling_on_sc=True),
    out_type=jax.ShapeDtypeStruct((B, D), jnp.float32),
    scratch_types=[
        pltpu.VMEM((SPW * F + 16,), jnp.int32),
        pltpu.VMEM((NBUF, F, 1, D), jnp.float32),
        pltpu.VMEM((SPW, D), jnp.float32),
        pltpu.SemaphoreType.DMA,
        pltpu.SemaphoreType.DMA,
    ],
)(_sc_bi_body)


BT = 1024


def _mlp_body(bi, xf, w1, b1, w2, b2, w3, wl, c0, out):
    h = jnp.maximum(
        jnp.dot(bi[...], w1[...], preferred_element_type=jnp.float32) + b1[...], 0.0
    )
    h = jnp.maximum(
        jnp.dot(h, w2[...], preferred_element_type=jnp.float32) + b2[...], 0.0
    )
    deep = jnp.dot(h, w3[...], preferred_element_type=jnp.float32)
    lin = jnp.dot(xf[...], wl[...], preferred_element_type=jnp.float32)
    out[...] = jax.nn.sigmoid(deep + lin + c0[...])


def _mlp_call(bi, xf, w1, b1, w2, b2, w3, wl, c0):
    grid = (B // BT,)
    return pl.pallas_call(
        _mlp_body,
        grid=grid,
        in_specs=[
            pl.BlockSpec((BT, D), lambda i: (i, 0)),
            pl.BlockSpec((BT, F), lambda i: (i, 0)),
            pl.BlockSpec((D, H1), lambda i: (0, 0)),
            pl.BlockSpec((1, H1), lambda i: (0, 0)),
            pl.BlockSpec((H1, H2), lambda i: (0, 0)),
            pl.BlockSpec((1, H2), lambda i: (0, 0)),
            pl.BlockSpec((H2, 1), lambda i: (0, 0)),
            pl.BlockSpec((F, 1), lambda i: (0, 0)),
            pl.BlockSpec((1, 1), lambda i: (0, 0)),
        ],
        out_specs=pl.BlockSpec((BT, 1), lambda i: (i, 0)),
        out_shape=jax.ShapeDtypeStruct((B, 1), jnp.float32),
    )(bi, xf, w1, b1, w2, b2, w3, wl, c0)


def kernel(x, tables, Wl, bl, W1, b1, W2, b2, W3, b3):
    x = x.astype(jnp.int32)
    bi = _sc_bi(x.reshape(B * F), tables)
    xf = x.astype(jnp.float32)
    c0 = (bl + b3).reshape(1, 1)
    return _mlp_call(
        bi,
        xf,
        W1,
        b1.reshape(1, H1),
        W2,
        b2.reshape(1, H2),
        W3,
        Wl,
        c0,
    )
